# baseline (speedup 1.0000x reference)
import functools

import jax
import jax.numpy as jnp
from jax import lax
from jax.experimental import pallas as pl
from jax.experimental.pallas import tpu as pltpu
from jax.experimental.pallas import tpu_sc as plsc

_BS, _NQ, _NC = 16, 300, 92
_NP = 304
_CH = _NP // 16
_BIG = 10000


def _cost_body(logits_ref, labels_ref, boxes_ref, tboxes_ref,
               cost_ref, cb_ref, slse_ref):
    cls_iota = jax.lax.broadcasted_iota(jnp.int32, (_NC, _NP), 0)
    lane1 = jax.lax.broadcasted_iota(jnp.int32, (1, _NP), 1)
    padmask = lane1 >= _NQ
    total_lse = jnp.float32(0.0)
    for b in range(_BS):
        x = logits_ref[b]
        m = jnp.max(x, axis=1, keepdims=True)
        e = jnp.exp(x - m)
        p = e / jnp.sum(e, axis=1, keepdims=True)
        m2 = jnp.max(p, axis=1, keepdims=True)
        lse2 = jnp.log(jnp.sum(jnp.exp(p - m2), axis=1, keepdims=True)) + m2
        total_lse = total_lse + jnp.sum(lse2)

        lab = labels_ref[b]
        onehot = (cls_iota == lab).astype(jnp.float32)
        g = jnp.dot(p, onehot, preferred_element_type=jnp.float32)
        cbcol = jnp.sum(jnp.abs(boxes_ref[b] - tboxes_ref[b]), axis=1,
                        keepdims=True)
        cbrow = jnp.concatenate(
            [jnp.transpose(cbcol), jnp.zeros((1, _NP - _NQ), jnp.float32)],
            axis=1)
        cb_ref[b] = cbrow
        cost_ref[b] = jnp.where(padmask, jnp.inf, 5.0 * cbrow - g)
    slse_ref[0, 0] = total_lse


@functools.cache
def _make_assign_kernel():
    mesh = plsc.VectorSubcoreMesh(core_axis_name="c", subcore_axis_name="s")
    return functools.partial(
        pl.kernel, mesh=mesh,
        out_type=[
            jax.ShapeDtypeStruct((_BS, 1, _NP), jnp.int32),
            jax.ShapeDtypeStruct((_BS, 16), jnp.float32),
        ],
        scratch_types=[
            pltpu.VMEM((_NQ, _NP), jnp.float32),
            pltpu.VMEM((1, _NP), jnp.int32),
            pltpu.VMEM((16,), jnp.float32),
            pltpu.SemaphoreType.DMA,
            pltpu.SemaphoreType.DMA,
        ],
    )(_assign_body)


_SPLIT = 64


def _assign_body(cost_hbm, inv_hbm, accm_hbm, cost_v, inv_v, accm_v,
                 sem1, sem2):
    b = lax.axis_index("s") * 2 + lax.axis_index("c")

    @pl.when(b < _BS)
    def _():
        cp1 = pltpu.async_copy(cost_hbm.at[b, pl.ds(0, _SPLIT)],
                               cost_v.at[pl.ds(0, _SPLIT)], sem1)
        cp2 = pltpu.async_copy(cost_hbm.at[b, pl.ds(_SPLIT, _NQ - _SPLIT)],
                               cost_v.at[pl.ds(_SPLIT, _NQ - _SPLIT)], sem2)
        cp1.wait()
        lanes = lax.iota(jnp.int32, 16)

        def allmin(x):
            for s in (8, 4, 2, 1):
                p = jnp.bitwise_xor(lanes, s)
                x = jnp.minimum(x, x.at[p].get(mode="promise_in_bounds"))
            return x

        def row(i, carry):
            acc_m = carry[0]
            invs = carry[1:]
            vals = [(jnp.where(invs[c] == _BIG, cost_v[i, pl.ds(c * 16, 16)],
                               jnp.inf), c * 16 + lanes)
                    for c in range(_CH)]
            while len(vals) > 1:
                nxt = []
                for k in range(0, len(vals) - 1, 2):
                    (va, ja), (vb, jb) = vals[k], vals[k + 1]
                    lt = vb < va
                    nxt.append((jnp.where(lt, vb, va), jnp.where(lt, jb, ja)))
                if len(vals) % 2:
                    nxt.append(vals[-1])
                vals = nxt
            best, bestj = vals[0]
            mval = allmin(best)
            jv = allmin(jnp.where(best == mval, bestj, 100000)
                        .astype(jnp.float32)).astype(jnp.int32)
            invs = tuple(jnp.where(c * 16 + lanes == jv, i, invs[c])
                         for c in range(_CH))
            return (acc_m + mval,) + invs

        big = jnp.full((16,), _BIG, jnp.int32)
        mid = lax.fori_loop(0, _SPLIT, row,
                            (jnp.zeros((16,), jnp.float32),)
                            + tuple(big for _ in range(_CH)), unroll=2)
        cp2.wait()
        fin = lax.fori_loop(_SPLIT, _NQ, row, mid, unroll=2)
        for c in range(_CH):
            inv_v[0, pl.ds(c * 16, 16)] = fin[1 + c]
        accm_v[...] = fin[0]
        pltpu.sync_copy(inv_v, inv_hbm.at[b])
        pltpu.sync_copy(accm_v, accm_hbm.at[b])


def _loss_body(inv_ref, accm_ref, cb_ref, slse_ref, boxes_ref, tboxes_ref,
               out_ref):
    row_iota = jax.lax.broadcasted_iota(jnp.int32, (_NQ, _NP), 0)
    lane1 = jax.lax.broadcasted_iota(jnp.int32, (1, _NP), 1)
    qmask = (lane1 < _NQ).astype(jnp.float32)
    total = slse_ref[0, 0] / jnp.float32(_NQ)
    for b in range(_BS):
        inv_row = inv_ref[b]
        psel = (row_iota == inv_row).astype(jnp.float32)
        sel = jnp.dot(psel, tboxes_ref[b],
                      preferred_element_type=jnp.float32)
        bb = jnp.sum(jnp.abs(boxes_ref[b] - sel))
        cbtot = jnp.sum(cb_ref[b] * qmask)
        total = total + (accm_ref[b, 0] - 5.0 * cbtot) / jnp.float32(_NQ) \
            + bb / jnp.float32(4 * _NQ)
    out_ref[0, 0] = total


def kernel(pred_logits, pred_boxes, tgt_labels, tgt_boxes):
    bs, nq, nc = pred_logits.shape
    pad = _NP - nq
    labels = tgt_labels.astype(jnp.int32).reshape(bs, 1, nq)
    labels = jnp.pad(labels, ((0, 0), (0, 0), (0, pad)), constant_values=nc)
    tboxes_pad = jnp.pad(tgt_boxes, ((0, 0), (0, pad), (0, 0)))
    cost, cb, slse = pl.pallas_call(
        _cost_body,
        out_shape=[
            jax.ShapeDtypeStruct((_BS, _NQ, _NP), jnp.float32),
            jax.ShapeDtypeStruct((_BS, 1, _NP), jnp.float32),
            jax.ShapeDtypeStruct((1, 1), jnp.float32),
        ],
        out_specs=[
            pl.BlockSpec((_BS, _NQ, _NP), lambda: (0, 0, 0)),
            pl.BlockSpec((_BS, 1, _NP), lambda: (0, 0, 0)),
            pl.BlockSpec(memory_space=pltpu.SMEM),
        ],
    )(pred_logits, labels, pred_boxes, tgt_boxes)
    inv, accm = _make_assign_kernel()(cost)
    out = pl.pallas_call(
        _loss_body,
        out_shape=jax.ShapeDtypeStruct((1, 1), jnp.float32),
        out_specs=pl.BlockSpec(memory_space=pltpu.SMEM),
        in_specs=[
            pl.BlockSpec((_BS, 1, _NP), lambda: (0, 0, 0)),
            pl.BlockSpec(memory_space=pltpu.SMEM),
            pl.BlockSpec((_BS, 1, _NP), lambda: (0, 0, 0)),
            pl.BlockSpec(memory_space=pltpu.SMEM),
            pl.BlockSpec((_BS, _NQ, 4), lambda: (0, 0, 0)),
            pl.BlockSpec((_BS, _NP, 4), lambda: (0, 0, 0)),
        ],
    )(inv, accm, cb, slse, pred_boxes, tboxes_pad)
    return out[0, 0]


# speedup vs baseline: 1.0211x; 1.0211x over previous
import functools

import jax
import jax.numpy as jnp
from jax import lax
from jax.experimental import pallas as pl
from jax.experimental.pallas import tpu as pltpu
from jax.experimental.pallas import tpu_sc as plsc

_BS, _NQ, _NC = 16, 300, 92
_NP = 304
_CH = _NP // 16
_BIG = 10000


def _cost_body(logits_ref, labels_ref, boxes_ref, tboxes_ref,
               cost_ref, cb_ref, slse_ref):
    cls_iota = jax.lax.broadcasted_iota(jnp.int32, (_NC, _NP), 0)
    lane1 = jax.lax.broadcasted_iota(jnp.int32, (1, _NP), 1)
    padmask = lane1 >= _NQ
    total_lse = jnp.float32(0.0)
    for b in range(_BS):
        x = logits_ref[b]
        m = jnp.max(x, axis=1, keepdims=True)
        e = jnp.exp(x - m)
        p = e / jnp.sum(e, axis=1, keepdims=True)
        m2 = jnp.max(p, axis=1, keepdims=True)
        lse2 = jnp.log(jnp.sum(jnp.exp(p - m2), axis=1, keepdims=True)) + m2
        total_lse = total_lse + jnp.sum(lse2)

        lab = labels_ref[b]
        onehot = (cls_iota == lab).astype(jnp.float32)
        g = jnp.dot(p, onehot, preferred_element_type=jnp.float32)
        cbcol = jnp.sum(jnp.abs(boxes_ref[b] - tboxes_ref[b]), axis=1,
                        keepdims=True)
        cbrow = jnp.concatenate(
            [jnp.transpose(cbcol), jnp.zeros((1, _NP - _NQ), jnp.float32)],
            axis=1)
        cb_ref[b] = cbrow
        cost_ref[b] = jnp.where(padmask, jnp.inf, 5.0 * cbrow - g)
    slse_ref[0, 0] = total_lse


@functools.cache
def _make_assign_kernel():
    mesh = plsc.VectorSubcoreMesh(core_axis_name="c", subcore_axis_name="s")
    return functools.partial(
        pl.kernel, mesh=mesh,
        out_type=[
            jax.ShapeDtypeStruct((_BS, 1, _NP), jnp.int32),
            jax.ShapeDtypeStruct((_BS, 16), jnp.float32),
        ],
        scratch_types=[
            pltpu.VMEM((_NQ, _NP), jnp.float32),
            pltpu.VMEM((1, _NP), jnp.int32),
            pltpu.VMEM((16,), jnp.float32),
            pltpu.SemaphoreType.DMA,
            pltpu.SemaphoreType.DMA,
        ],
    )(_assign_body)


_SPLIT = 64


def _assign_body(cost_hbm, inv_hbm, accm_hbm, cost_v, inv_v, accm_v,
                 sem1, sem2):
    b = lax.axis_index("s") * 2 + lax.axis_index("c")

    @pl.when(b < _BS)
    def _():
        cp1 = pltpu.async_copy(cost_hbm.at[b, pl.ds(0, _SPLIT)],
                               cost_v.at[pl.ds(0, _SPLIT)], sem1)
        cp2 = pltpu.async_copy(cost_hbm.at[b, pl.ds(_SPLIT, _NQ - _SPLIT)],
                               cost_v.at[pl.ds(_SPLIT, _NQ - _SPLIT)], sem2)
        cp1.wait()
        lanes = lax.iota(jnp.int32, 16)

        def allmin(x):
            for s in (8, 4, 2, 1):
                p = jnp.bitwise_xor(lanes, s)
                x = jnp.minimum(x, x.at[p].get(mode="promise_in_bounds"))
            return x

        def row(i, carry):
            acc_m = carry[0]
            invs = carry[1:]
            vals = [(jnp.where(invs[c] == _BIG, cost_v[i, pl.ds(c * 16, 16)],
                               jnp.inf), c * 16 + lanes)
                    for c in range(_CH)]
            while len(vals) > 1:
                nxt = []
                for k in range(0, len(vals) - 1, 2):
                    (va, ja), (vb, jb) = vals[k], vals[k + 1]
                    lt = vb < va
                    nxt.append((jnp.where(lt, vb, va), jnp.where(lt, jb, ja)))
                if len(vals) % 2:
                    nxt.append(vals[-1])
                vals = nxt
            best, bestj = vals[0]
            mval = allmin(best)
            jv = allmin(jnp.where(best == mval, bestj, 100000)
                        .astype(jnp.float32)).astype(jnp.int32)
            invs = tuple(jnp.where(c * 16 + lanes == jv, i, invs[c])
                         for c in range(_CH))
            return (acc_m + mval,) + invs

        big = jnp.full((16,), _BIG, jnp.int32)
        mid = lax.fori_loop(0, _SPLIT, row,
                            (jnp.zeros((16,), jnp.float32),)
                            + tuple(big for _ in range(_CH)))
        cp2.wait()
        fin = lax.fori_loop(_SPLIT, _NQ, row, mid)
        for c in range(_CH):
            inv_v[0, pl.ds(c * 16, 16)] = fin[1 + c]
        accm_v[...] = fin[0]
        pltpu.sync_copy(inv_v, inv_hbm.at[b])
        pltpu.sync_copy(accm_v, accm_hbm.at[b])


def _loss_body(inv_ref, accm_ref, cb_ref, slse_ref, boxes_ref, tboxes_ref,
               out_ref):
    row_iota = jax.lax.broadcasted_iota(jnp.int32, (_NQ, _NP), 0)
    lane1 = jax.lax.broadcasted_iota(jnp.int32, (1, _NP), 1)
    qmask = (lane1 < _NQ).astype(jnp.float32)
    total = slse_ref[0, 0] / jnp.float32(_NQ)
    for b in range(_BS):
        inv_row = inv_ref[b]
        psel = (row_iota == inv_row).astype(jnp.float32)
        sel = jnp.dot(psel, tboxes_ref[b],
                      preferred_element_type=jnp.float32)
        bb = jnp.sum(jnp.abs(boxes_ref[b] - sel))
        cbtot = jnp.sum(cb_ref[b] * qmask)
        total = total + (accm_ref[b, 0] - 5.0 * cbtot) / jnp.float32(_NQ) \
            + bb / jnp.float32(4 * _NQ)
    out_ref[0, 0] = total


def kernel(pred_logits, pred_boxes, tgt_labels, tgt_boxes):
    bs, nq, nc = pred_logits.shape
    pad = _NP - nq
    labels = tgt_labels.astype(jnp.int32).reshape(bs, 1, nq)
    labels = jnp.pad(labels, ((0, 0), (0, 0), (0, pad)), constant_values=nc)
    tboxes_pad = jnp.pad(tgt_boxes, ((0, 0), (0, pad), (0, 0)))
    cost, cb, slse = pl.pallas_call(
        _cost_body,
        out_shape=[
            jax.ShapeDtypeStruct((_BS, _NQ, _NP), jnp.float32),
            jax.ShapeDtypeStruct((_BS, 1, _NP), jnp.float32),
            jax.ShapeDtypeStruct((1, 1), jnp.float32),
        ],
        out_specs=[
            pl.BlockSpec((_BS, _NQ, _NP), lambda: (0, 0, 0)),
            pl.BlockSpec((_BS, 1, _NP), lambda: (0, 0, 0)),
            pl.BlockSpec(memory_space=pltpu.SMEM),
        ],
    )(pred_logits, labels, pred_boxes, tgt_boxes)
    inv, accm = _make_assign_kernel()(cost)
    out = pl.pallas_call(
        _loss_body,
        out_shape=jax.ShapeDtypeStruct((1, 1), jnp.float32),
        out_specs=pl.BlockSpec(memory_space=pltpu.SMEM),
        in_specs=[
            pl.BlockSpec((_BS, 1, _NP), lambda: (0, 0, 0)),
            pl.BlockSpec(memory_space=pltpu.SMEM),
            pl.BlockSpec((_BS, 1, _NP), lambda: (0, 0, 0)),
            pl.BlockSpec(memory_space=pltpu.SMEM),
            pl.BlockSpec((_BS, _NQ, 4), lambda: (0, 0, 0)),
            pl.BlockSpec((_BS, _NP, 4), lambda: (0, 0, 0)),
        ],
    )(inv, accm, cb, slse, pred_boxes, tboxes_pad)
    return out[0, 0]
